# baseline (speedup 1.0000x reference)
import functools

import jax
import jax.numpy as jnp
from jax import lax
from jax.experimental import pallas as pl
from jax.experimental.pallas import tpu as pltpu
from jax.experimental.pallas import tpu_sc as plsc


def _routing_body(x_ref, rc_ref, gw_ref, cw_ref, pe_ref, po_ref, wa_ref,
                  wb_ref, cnt_ref, *, NW):
    T, E = x_ref.shape[0], gw_ref.shape[0]
    TPW = T // NW
    ctx = jax.lax.dot_general(
        rc_ref[...].astype(jnp.bfloat16),
        cw_ref[...].astype(jnp.bfloat16), (((1,), (1,)), ((), ())),
        preferred_element_type=jnp.float32)
    xr = x_ref[...] + ctx
    logits = jax.lax.dot_general(
        xr.astype(jnp.bfloat16), gw_ref[...].astype(jnp.bfloat16),
        (((1,), (1,)), ((), ())),
        preferred_element_type=jnp.float32)
    iota_e = jax.lax.broadcasted_iota(jnp.int32, (T, E), 1)
    m1 = jnp.max(logits, axis=1, keepdims=True)
    a1 = jnp.min(jnp.where(logits == m1, iota_e, E), axis=1,
                 keepdims=True).astype(jnp.int32)
    neg = jnp.where(iota_e == a1, -jnp.inf, logits)
    m2 = jnp.max(neg, axis=1, keepdims=True)
    a2 = jnp.min(jnp.where(neg == m2, iota_e, E), axis=1,
                 keepdims=True).astype(jnp.int32)
    d = jnp.exp(m2 - m1)
    wa_ref[...] = 1.0 / (1.0 + d)
    wb_ref[...] = d / (1.0 + d)

    o1 = (iota_e == a1).astype(jnp.int32)
    o2 = (iota_e == a2).astype(jnp.int32)
    cnt = o1 + o2
    cnt_ref[...] = cnt.sum(axis=0, keepdims=True)

    tpos = jax.lax.broadcasted_iota(jnp.int32, (T, E), 0) % TPW
    s = cnt
    k = 1
    while k < TPW:
        sh = jnp.concatenate([jnp.zeros((k, E), jnp.int32), s[:-k]], axis=0)
        s = s + jnp.where(tpos >= k, sh, 0)
        k *= 2
    sexc = s - cnt

    hist = cnt.reshape(NW, TPW, E).sum(axis=1)
    h = hist
    k = 1
    while k < NW:
        h = h + jnp.concatenate(
            [jnp.zeros((k, E), jnp.int32), h[:-k]], axis=0)
        k *= 2
    wexc = h - hist

    counts = hist.sum(axis=0, keepdims=True)
    c = counts
    k = 1
    while k < E:
        c = c + jnp.concatenate(
            [jnp.zeros((1, k), jnp.int32), c[:, :-k]], axis=1)
        k *= 2
    offs = c - counts

    start = wexc + offs
    base = jnp.broadcast_to(
        start.reshape(NW, 1, E), (NW, TPW, E)).reshape(T, E)
    slot = base + sexc
    pe_ref[...] = jnp.sum(slot * o1, axis=1, keepdims=True)
    po_ref[...] = jnp.sum(slot * o2, axis=1, keepdims=True)


def _routing(x_flat, routing_context, gate_W, ctx_W, NW):
    T = x_flat.shape[0]
    E = gate_W.shape[0]
    out_shape = [
        jax.ShapeDtypeStruct((T, 1), jnp.int32),
        jax.ShapeDtypeStruct((T, 1), jnp.int32),
        jax.ShapeDtypeStruct((T, 1), jnp.float32),
        jax.ShapeDtypeStruct((T, 1), jnp.float32),
        jax.ShapeDtypeStruct((1, E), jnp.int32),
    ]
    return pl.pallas_call(
        functools.partial(_routing_body, NW=NW), out_shape=out_shape)(
            x_flat, routing_context, gate_W, ctx_W)


def _gmm_body(tile_r, exp_r, lo_r, hi_r, *refs, M, KH):
    x_ref = refs[0]
    b2_ref = refs[1 + 3 * KH]
    ws_ref = refs[2 + 3 * KH]
    o_ref = refs[3 + 3 * KH]
    l = pl.program_id(0)
    x = x_ref[...].astype(jnp.bfloat16)
    y = None
    for i in range(KH):
        w1_ref = refs[1 + 3 * i]
        b1_ref = refs[2 + 3 * i]
        w2_ref = refs[3 + 3 * i]
        h = jnp.dot(x, w1_ref[0].astype(jnp.bfloat16),
                    preferred_element_type=jnp.float32)
        h = h + b1_ref[0, 0][None, :]
        h = 0.5 * h * (1.0 + jax.lax.erf(h * 0.7071067811865476))
        yi = jnp.dot(h.astype(jnp.bfloat16), w2_ref[0].astype(jnp.bfloat16),
                     preferred_element_type=jnp.float32)
        y = yi if y is None else y + yi
    y = y + b2_ref[0, 0][None, :]
    row = tile_r[l] * M + jax.lax.broadcasted_iota(jnp.int32, (M, 1), 0)
    valid = (row >= lo_r[l]) & (row < hi_r[l])
    scale = jnp.where(valid, ws_ref[0, 0][:, None], 0.0)
    contrib = y * scale
    first = jnp.logical_or(l == 0, tile_r[l] != tile_r[jnp.maximum(l - 1, 0)])

    @pl.when(first)
    def _():
        o_ref[...] = contrib

    @pl.when(jnp.logical_not(first))
    def _():
        o_ref[...] += contrib


def _gmm(x_sorted, w_sorted3, W1, b1, W2, b2, tile_of, expert_of, row_lo,
         row_hi, M, KH=4):
    TK, C = x_sorted.shape
    E, _, H = W1.shape
    HC = H // KH
    G = tile_of.shape[0]
    b1r = b1.reshape(E, 1, H)
    b2r = b2.reshape(E, 1, C)
    in_specs = [pl.BlockSpec((M, C), lambda l, t, e, lo, hi: (t[l], 0))]
    operands = [x_sorted]
    for i in range(KH):
        in_specs += [
            pl.BlockSpec((1, C, HC),
                         lambda l, t, e, lo, hi, i=i: (e[l], 0, i)),
            pl.BlockSpec((1, 1, HC),
                         lambda l, t, e, lo, hi, i=i: (e[l], 0, i)),
            pl.BlockSpec((1, HC, C),
                         lambda l, t, e, lo, hi, i=i: (e[l], i, 0)),
        ]
        operands += [W1, b1r, W2]
    in_specs += [
        pl.BlockSpec((1, 1, C), lambda l, t, e, lo, hi: (e[l], 0, 0)),
        pl.BlockSpec((1, 1, M), lambda l, t, e, lo, hi: (t[l], 0, 0)),
    ]
    operands += [b2r, w_sorted3]
    grid_spec = pltpu.PrefetchScalarGridSpec(
        num_scalar_prefetch=4,
        grid=(G,),
        in_specs=in_specs,
        out_specs=pl.BlockSpec((M, C), lambda l, t, e, lo, hi: (t[l], 0)),
    )
    return pl.pallas_call(
        functools.partial(_gmm_body, M=M, KH=KH),
        grid_spec=grid_spec,
        out_shape=jax.ShapeDtypeStruct((TK, C), jnp.float32),
        compiler_params=pltpu.CompilerParams(
            dimension_semantics=("arbitrary",)),
    )(tile_of, expert_of, row_lo, row_hi, *operands)


def _sc_mesh():
    return plsc.VectorSubcoreMesh(core_axis_name="c", subcore_axis_name="s")


def _sc_wid():
    info = plsc.get_sparse_core_info()
    return lax.axis_index("s") * info.num_cores + lax.axis_index("c")


_LANE0 = None


def _lane0():
    return lax.iota(jnp.int32, 16) == 0


def _sc_dispatch(pe, po, wa, wb, x_flat, NW):
    T, C = x_flat.shape
    TK = 2 * T
    TPW = T // NW

    out_type = [
        jax.ShapeDtypeStruct((TK,), jnp.float32),
        jax.ShapeDtypeStruct((TK, C), jnp.float32),
    ]

    @functools.partial(
        pl.kernel,
        out_type=out_type,
        mesh=_sc_mesh(),
        scratch_types=[
            pltpu.VMEM((TPW,), jnp.int32),
            pltpu.VMEM((TPW,), jnp.int32),
            pltpu.VMEM((TPW,), jnp.float32),
            pltpu.VMEM((TPW,), jnp.float32),
            pltpu.VMEM((TPW, C), jnp.float32),
            pltpu.SemaphoreType.DMA,
        ],
    )
    def dispatch_kernel(pe_hbm, po_hbm, wa_hbm, wb_hbm, x_hbm,
                        ws_hbm, xs_hbm,
                        pe_v, po_v, wa_v, wb_v, x_v, sem):
        wid = _sc_wid()
        tbase = wid * TPW
        pltpu.sync_copy(pe_hbm.at[pl.ds(tbase, TPW)], pe_v)
        pltpu.sync_copy(po_hbm.at[pl.ds(tbase, TPW)], po_v)
        pltpu.sync_copy(wa_hbm.at[pl.ds(tbase, TPW)], wa_v)
        pltpu.sync_copy(wb_hbm.at[pl.ds(tbase, TPW)], wb_v)
        pltpu.sync_copy(x_hbm.at[pl.ds(tbase, TPW)], x_v)
        c1 = pltpu.async_copy(wa_v, ws_hbm.at[pe_v], sem)
        c2 = pltpu.async_copy(wb_v, ws_hbm.at[po_v], sem)
        c3 = pltpu.async_copy(x_v, xs_hbm.at[pe_v], sem)
        c4 = pltpu.async_copy(x_v, xs_hbm.at[po_v], sem)
        c1.wait()
        c2.wait()
        c3.wait()
        c4.wait()

    return dispatch_kernel(pe, po, wa, wb, x_flat)


def _sc_combine(y_sorted, pe, po, NW):
    T = pe.shape[0]
    C = y_sorted.shape[1]
    TPW = T // NW

    @functools.partial(
        pl.kernel,
        out_type=jax.ShapeDtypeStruct((T, C), jnp.float32),
        mesh=_sc_mesh(),
        scratch_types=[
            pltpu.VMEM((TPW,), jnp.int32),
            pltpu.VMEM((TPW,), jnp.int32),
            pltpu.VMEM((TPW, C), jnp.float32),
            pltpu.VMEM((TPW, C), jnp.float32),
            pltpu.SemaphoreType.DMA,
        ],
    )
    def combine_kernel(y_hbm, pe_hbm, po_hbm, out_hbm,
                       pe_v, po_v, ya_v, yb_v, sem):
        wid = _sc_wid()
        tbase = wid * TPW
        pltpu.sync_copy(pe_hbm.at[pl.ds(tbase, TPW)], pe_v)
        pltpu.sync_copy(po_hbm.at[pl.ds(tbase, TPW)], po_v)
        ca = pltpu.async_copy(y_hbm.at[pe_v], ya_v, sem)
        cb = pltpu.async_copy(y_hbm.at[po_v], yb_v, sem)
        ca.wait()
        cb.wait()

        def body(r, carry):
            for c in range(C // 16):
                sl = pl.ds(c * 16, 16)
                ya_v[r, sl] = ya_v[r, sl] + yb_v[r, sl]
            return carry

        lax.fori_loop(0, TPW, body, 0)
        pltpu.sync_copy(ya_v, out_hbm.at[pl.ds(tbase, TPW)])

    return combine_kernel(y_sorted, pe, po)


def _block_metadata(offsets, E, NT, M):
    G = NT + E - 1
    counts = offsets[1:] - offsets[:-1]
    t_start = offsets[:-1] // M
    t_end = (offsets[1:] + M - 1) // M
    ntpe = jnp.where(counts > 0, t_end - t_start, 0)
    cum_v = jnp.cumsum(ntpe)
    cum_before = cum_v - ntpe
    l = jnp.arange(G, dtype=jnp.int32)
    eid = jnp.searchsorted(cum_v, l, side="right").astype(jnp.int32)
    eidc = jnp.minimum(eid, E - 1)
    valid = l < cum_v[-1]
    tile_of = jnp.clip(t_start[eidc] + (l - cum_before[eidc]), 0, NT - 1)
    row_lo = jnp.where(valid, offsets[eidc], 0)
    row_hi = jnp.where(valid, offsets[eidc + 1], 0)
    return (tile_of.astype(jnp.int32), eidc.astype(jnp.int32),
            row_lo.astype(jnp.int32), row_hi.astype(jnp.int32))


def kernel(x, routing_context, gate_W, ctx_W, W1, b1, W2, b2):
    B, N, C = x.shape
    E, _, H = W1.shape
    K = 2
    T = B * N
    TK = T * K
    M = 64
    NT = TK // M

    info = plsc.get_sparse_core_info()
    NW = info.num_cores * info.num_subcores

    x_flat = x.reshape(T, C)
    pe, po, wa, wb, counts = _routing(
        x_flat, routing_context, gate_W, ctx_W, NW)
    pe = pe.reshape(T)
    po = po.reshape(T)

    offsets = jnp.concatenate(
        [jnp.zeros((1,), jnp.int32),
         jnp.cumsum(counts[0]).astype(jnp.int32)])
    tile_of, expert_of, row_lo, row_hi = _block_metadata(offsets, E, NT, M)

    ws, x_sorted = _sc_dispatch(
        pe, po, wa.reshape(T), wb.reshape(T), x_flat, NW)

    y_sorted = _gmm(x_sorted, ws.reshape(NT, 1, M), W1, b1, W2, b2,
                    tile_of, expert_of, row_lo, row_hi, M, KH=4)

    out_flat = _sc_combine(y_sorted, pe, po, NW)
    return out_flat.reshape(B, N, C)


# speedup vs baseline: 1.1255x; 1.1255x over previous
import functools

import jax
import jax.numpy as jnp
from jax import lax
from jax.experimental import pallas as pl
from jax.experimental.pallas import tpu as pltpu
from jax.experimental.pallas import tpu_sc as plsc


def _routing_body(x_ref, rc_ref, gw_ref, cw_ref, pe_ref, po_ref, wa_ref,
                  wb_ref, cnt_ref, *, NW):
    T, E = x_ref.shape[0], gw_ref.shape[0]
    TPW = T // NW
    ctx = jax.lax.dot_general(
        rc_ref[...].astype(jnp.bfloat16),
        cw_ref[...].astype(jnp.bfloat16), (((1,), (1,)), ((), ())),
        preferred_element_type=jnp.float32)
    xr = x_ref[...] + ctx
    logits = jax.lax.dot_general(
        xr.astype(jnp.bfloat16), gw_ref[...].astype(jnp.bfloat16),
        (((1,), (1,)), ((), ())),
        preferred_element_type=jnp.float32)
    iota_e = jax.lax.broadcasted_iota(jnp.int32, (T, E), 1)
    m1 = jnp.max(logits, axis=1, keepdims=True)
    a1 = jnp.min(jnp.where(logits == m1, iota_e, E), axis=1,
                 keepdims=True).astype(jnp.int32)
    neg = jnp.where(iota_e == a1, -jnp.inf, logits)
    m2 = jnp.max(neg, axis=1, keepdims=True)
    a2 = jnp.min(jnp.where(neg == m2, iota_e, E), axis=1,
                 keepdims=True).astype(jnp.int32)
    d = jnp.exp(m2 - m1)
    wa_ref[...] = 1.0 / (1.0 + d)
    wb_ref[...] = d / (1.0 + d)

    o1 = (iota_e == a1).astype(jnp.int32)
    o2 = (iota_e == a2).astype(jnp.int32)
    cnt = o1 + o2
    cnt_ref[...] = cnt.sum(axis=0, keepdims=True)

    tpos = jax.lax.broadcasted_iota(jnp.int32, (T, E), 0) % TPW
    s = cnt
    k = 1
    while k < TPW:
        sh = jnp.concatenate([jnp.zeros((k, E), jnp.int32), s[:-k]], axis=0)
        s = s + jnp.where(tpos >= k, sh, 0)
        k *= 2
    sexc = s - cnt

    hist = cnt.reshape(NW, TPW, E).sum(axis=1)
    h = hist
    k = 1
    while k < NW:
        h = h + jnp.concatenate(
            [jnp.zeros((k, E), jnp.int32), h[:-k]], axis=0)
        k *= 2
    wexc = h - hist

    counts = hist.sum(axis=0, keepdims=True)
    c = counts
    k = 1
    while k < E:
        c = c + jnp.concatenate(
            [jnp.zeros((1, k), jnp.int32), c[:, :-k]], axis=1)
        k *= 2
    offs = c - counts

    start = wexc + offs
    base = jnp.broadcast_to(
        start.reshape(NW, 1, E), (NW, TPW, E)).reshape(T, E)
    slot = base + sexc
    pe_ref[...] = jnp.sum(slot * o1, axis=1, keepdims=True)
    po_ref[...] = jnp.sum(slot * o2, axis=1, keepdims=True)


def _routing(x_flat, routing_context, gate_W, ctx_W, NW):
    T = x_flat.shape[0]
    E = gate_W.shape[0]
    out_shape = [
        jax.ShapeDtypeStruct((T, 1), jnp.int32),
        jax.ShapeDtypeStruct((T, 1), jnp.int32),
        jax.ShapeDtypeStruct((T, 1), jnp.float32),
        jax.ShapeDtypeStruct((T, 1), jnp.float32),
        jax.ShapeDtypeStruct((1, E), jnp.int32),
    ]
    return pl.pallas_call(
        functools.partial(_routing_body, NW=NW), out_shape=out_shape)(
            x_flat, routing_context, gate_W, ctx_W)


def _gmm_body(tile_r, exp_r, lo_r, hi_r, *refs, M, KH):
    x_ref = refs[0]
    b2_ref = refs[1 + 3 * KH]
    ws_ref = refs[2 + 3 * KH]
    o_ref = refs[3 + 3 * KH]
    l = pl.program_id(0)
    x = x_ref[...].astype(jnp.bfloat16)
    y = None
    for i in range(KH):
        w1_ref = refs[1 + 3 * i]
        b1_ref = refs[2 + 3 * i]
        w2_ref = refs[3 + 3 * i]
        h = jnp.dot(x, w1_ref[0].astype(jnp.bfloat16),
                    preferred_element_type=jnp.float32)
        h = h + b1_ref[0, 0][None, :]
        h = 0.5 * h * (1.0 + jax.lax.erf(h * 0.7071067811865476))
        yi = jnp.dot(h.astype(jnp.bfloat16), w2_ref[0].astype(jnp.bfloat16),
                     preferred_element_type=jnp.float32)
        y = yi if y is None else y + yi
    y = y + b2_ref[0, 0][None, :]
    row = tile_r[l] * M + jax.lax.broadcasted_iota(jnp.int32, (M, 1), 0)
    valid = (row >= lo_r[l]) & (row < hi_r[l])
    scale = jnp.where(valid, ws_ref[0, 0][:, None], 0.0)
    contrib = y * scale
    first = jnp.logical_or(l == 0, tile_r[l] != tile_r[jnp.maximum(l - 1, 0)])

    @pl.when(first)
    def _():
        o_ref[...] = contrib

    @pl.when(jnp.logical_not(first))
    def _():
        o_ref[...] += contrib


def _gmm(x_sorted, w_sorted3, W1, b1, W2, b2, tile_of, expert_of, row_lo,
         row_hi, M, KH=4):
    TK, C = x_sorted.shape
    E, _, H = W1.shape
    HC = H // KH
    G = tile_of.shape[0]
    b1r = b1.reshape(E, 1, H)
    b2r = b2.reshape(E, 1, C)
    in_specs = [pl.BlockSpec((M, C), lambda l, t, e, lo, hi: (t[l], 0))]
    operands = [x_sorted]
    for i in range(KH):
        in_specs += [
            pl.BlockSpec((1, C, HC),
                         lambda l, t, e, lo, hi, i=i: (e[l], 0, i)),
            pl.BlockSpec((1, 1, HC),
                         lambda l, t, e, lo, hi, i=i: (e[l], 0, i)),
            pl.BlockSpec((1, HC, C),
                         lambda l, t, e, lo, hi, i=i: (e[l], i, 0)),
        ]
        operands += [W1, b1r, W2]
    in_specs += [
        pl.BlockSpec((1, 1, C), lambda l, t, e, lo, hi: (e[l], 0, 0)),
        pl.BlockSpec((1, 1, M), lambda l, t, e, lo, hi: (t[l], 0, 0)),
    ]
    operands += [b2r, w_sorted3]
    grid_spec = pltpu.PrefetchScalarGridSpec(
        num_scalar_prefetch=4,
        grid=(G,),
        in_specs=in_specs,
        out_specs=pl.BlockSpec((M, C), lambda l, t, e, lo, hi: (t[l], 0)),
    )
    return pl.pallas_call(
        functools.partial(_gmm_body, M=M, KH=KH),
        grid_spec=grid_spec,
        out_shape=jax.ShapeDtypeStruct((TK, C), jnp.float32),
        compiler_params=pltpu.CompilerParams(
            dimension_semantics=("arbitrary",)),
    )(tile_of, expert_of, row_lo, row_hi, *operands)


def _sc_mesh():
    return plsc.VectorSubcoreMesh(core_axis_name="c", subcore_axis_name="s")


def _sc_wid():
    info = plsc.get_sparse_core_info()
    return lax.axis_index("s") * info.num_cores + lax.axis_index("c")


_LANE0 = None


def _lane0():
    return lax.iota(jnp.int32, 16) == 0


def _sc_dispatch(pe, po, wa, wb, x_flat, NW):
    T, C = x_flat.shape
    TK = 2 * T
    TPW = T // NW

    out_type = [
        jax.ShapeDtypeStruct((TK,), jnp.float32),
        jax.ShapeDtypeStruct((TK, C), jnp.float32),
    ]

    @functools.partial(
        pl.kernel,
        out_type=out_type,
        mesh=_sc_mesh(),
        scratch_types=[
            pltpu.VMEM((TPW,), jnp.int32),
            pltpu.VMEM((TPW,), jnp.int32),
            pltpu.VMEM((TPW,), jnp.float32),
            pltpu.VMEM((TPW,), jnp.float32),
            pltpu.VMEM((TPW, C), jnp.float32),
            pltpu.SemaphoreType.DMA,
        ],
    )
    def dispatch_kernel(pe_hbm, po_hbm, wa_hbm, wb_hbm, x_hbm,
                        ws_hbm, xs_hbm,
                        pe_v, po_v, wa_v, wb_v, x_v, sem):
        wid = _sc_wid()
        tbase = wid * TPW
        pltpu.sync_copy(pe_hbm.at[pl.ds(tbase, TPW)], pe_v)
        pltpu.sync_copy(po_hbm.at[pl.ds(tbase, TPW)], po_v)
        pltpu.sync_copy(wa_hbm.at[pl.ds(tbase, TPW)], wa_v)
        pltpu.sync_copy(wb_hbm.at[pl.ds(tbase, TPW)], wb_v)
        pltpu.sync_copy(x_hbm.at[pl.ds(tbase, TPW)], x_v)
        c1 = pltpu.async_copy(wa_v, ws_hbm.at[pe_v], sem)
        c2 = pltpu.async_copy(wb_v, ws_hbm.at[po_v], sem)
        c3 = pltpu.async_copy(x_v, xs_hbm.at[pe_v], sem)
        c4 = pltpu.async_copy(x_v, xs_hbm.at[po_v], sem)
        c1.wait()
        c2.wait()
        c3.wait()
        c4.wait()

    return dispatch_kernel(pe, po, wa, wb, x_flat)


def _sc_combine(y_sorted, pe, po, NW):
    T = pe.shape[0]
    C = y_sorted.shape[1]
    TPW = T // NW

    @functools.partial(
        pl.kernel,
        out_type=jax.ShapeDtypeStruct((T, C), jnp.float32),
        mesh=_sc_mesh(),
        scratch_types=[
            pltpu.VMEM((TPW,), jnp.int32),
            pltpu.VMEM((TPW,), jnp.int32),
            pltpu.VMEM((TPW, C), jnp.float32),
            pltpu.VMEM((TPW, C), jnp.float32),
            pltpu.SemaphoreType.DMA,
        ],
    )
    def combine_kernel(y_hbm, pe_hbm, po_hbm, out_hbm,
                       pe_v, po_v, ya_v, yb_v, sem):
        wid = _sc_wid()
        tbase = wid * TPW
        pltpu.sync_copy(pe_hbm.at[pl.ds(tbase, TPW)], pe_v)
        pltpu.sync_copy(po_hbm.at[pl.ds(tbase, TPW)], po_v)
        ca = pltpu.async_copy(y_hbm.at[pe_v], ya_v, sem)
        cb = pltpu.async_copy(y_hbm.at[po_v], yb_v, sem)
        ca.wait()
        cb.wait()

        def body(r, carry):
            for c in range(C // 16):
                sl = pl.ds(c * 16, 16)
                ya_v[r, sl] = ya_v[r, sl] + yb_v[r, sl]
            return carry

        lax.fori_loop(0, TPW, body, 0)
        pltpu.sync_copy(ya_v, out_hbm.at[pl.ds(tbase, TPW)])

    return combine_kernel(y_sorted, pe, po)


def _block_metadata(offsets, E, NT, M):
    G = NT + E - 1
    counts = offsets[1:] - offsets[:-1]
    t_start = offsets[:-1] // M
    t_end = (offsets[1:] + M - 1) // M
    ntpe = jnp.where(counts > 0, t_end - t_start, 0)
    cum_v = jnp.cumsum(ntpe)
    cum_before = cum_v - ntpe
    l = jnp.arange(G, dtype=jnp.int32)
    eid = jnp.searchsorted(cum_v, l, side="right").astype(jnp.int32)
    eidc = jnp.minimum(eid, E - 1)
    valid = l < cum_v[-1]
    tile_of = jnp.clip(t_start[eidc] + (l - cum_before[eidc]), 0, NT - 1)
    row_lo = jnp.where(valid, offsets[eidc], 0)
    row_hi = jnp.where(valid, offsets[eidc + 1], 0)
    return (tile_of.astype(jnp.int32), eidc.astype(jnp.int32),
            row_lo.astype(jnp.int32), row_hi.astype(jnp.int32))


def kernel(x, routing_context, gate_W, ctx_W, W1, b1, W2, b2):
    B, N, C = x.shape
    E, _, H = W1.shape
    K = 2
    T = B * N
    TK = T * K
    M = 128
    NT = TK // M

    info = plsc.get_sparse_core_info()
    NW = info.num_cores * info.num_subcores

    x_flat = x.reshape(T, C)
    pe, po, wa, wb, counts = _routing(
        x_flat, routing_context, gate_W, ctx_W, NW)
    pe = pe.reshape(T)
    po = po.reshape(T)

    offsets = jnp.concatenate(
        [jnp.zeros((1,), jnp.int32),
         jnp.cumsum(counts[0]).astype(jnp.int32)])
    tile_of, expert_of, row_lo, row_hi = _block_metadata(offsets, E, NT, M)

    ws, x_sorted = _sc_dispatch(
        pe, po, wa.reshape(T), wb.reshape(T), x_flat, NW)

    y_sorted = _gmm(x_sorted, ws.reshape(NT, 1, M), W1, b1, W2, b2,
                    tile_of, expert_of, row_lo, row_hi, M, KH=1)

    out_flat = _sc_combine(y_sorted, pe, po, NW)
    return out_flat.reshape(B, N, C)


# speedup vs baseline: 1.2033x; 1.0691x over previous
import functools

import jax
import jax.numpy as jnp
from jax import lax
from jax.experimental import pallas as pl
from jax.experimental.pallas import tpu as pltpu
from jax.experimental.pallas import tpu_sc as plsc


def _routing_body(x_ref, rc_ref, gw_ref, cw_ref, pe_ref, po_ref, wa_ref,
                  wb_ref, cnt_ref, *, NW):
    T, E = x_ref.shape[0], gw_ref.shape[0]
    TPW = T // NW
    ctx = jax.lax.dot_general(
        rc_ref[...].astype(jnp.bfloat16),
        cw_ref[...].astype(jnp.bfloat16), (((1,), (1,)), ((), ())),
        preferred_element_type=jnp.float32)
    xr = x_ref[...] + ctx
    logits = jax.lax.dot_general(
        xr.astype(jnp.bfloat16), gw_ref[...].astype(jnp.bfloat16),
        (((1,), (1,)), ((), ())),
        preferred_element_type=jnp.float32)
    iota_e = jax.lax.broadcasted_iota(jnp.int32, (T, E), 1)
    m1 = jnp.max(logits, axis=1, keepdims=True)
    a1 = jnp.min(jnp.where(logits == m1, iota_e, E), axis=1,
                 keepdims=True).astype(jnp.int32)
    neg = jnp.where(iota_e == a1, -jnp.inf, logits)
    m2 = jnp.max(neg, axis=1, keepdims=True)
    a2 = jnp.min(jnp.where(neg == m2, iota_e, E), axis=1,
                 keepdims=True).astype(jnp.int32)
    d = jnp.exp(m2 - m1)
    wa_ref[...] = 1.0 / (1.0 + d)
    wb_ref[...] = d / (1.0 + d)

    o1 = (iota_e == a1).astype(jnp.int32)
    o2 = (iota_e == a2).astype(jnp.int32)
    cnt = o1 + o2
    cnt_ref[...] = cnt.sum(axis=0, keepdims=True)

    tpos = jax.lax.broadcasted_iota(jnp.int32, (T, E), 0) % TPW
    s = cnt
    k = 1
    while k < TPW:
        sh = jnp.concatenate([jnp.zeros((k, E), jnp.int32), s[:-k]], axis=0)
        s = s + jnp.where(tpos >= k, sh, 0)
        k *= 2
    sexc = s - cnt

    hist = cnt.reshape(NW, TPW, E).sum(axis=1)
    h = hist
    k = 1
    while k < NW:
        h = h + jnp.concatenate(
            [jnp.zeros((k, E), jnp.int32), h[:-k]], axis=0)
        k *= 2
    wexc = h - hist

    counts = hist.sum(axis=0, keepdims=True)
    c = counts
    k = 1
    while k < E:
        c = c + jnp.concatenate(
            [jnp.zeros((1, k), jnp.int32), c[:, :-k]], axis=1)
        k *= 2
    offs = c - counts

    start = wexc + offs
    base = jnp.broadcast_to(
        start.reshape(NW, 1, E), (NW, TPW, E)).reshape(T, E)
    slot = base + sexc
    pe_ref[...] = jnp.sum(slot * o1, axis=1, keepdims=True)
    po_ref[...] = jnp.sum(slot * o2, axis=1, keepdims=True)


def _routing(x_flat, routing_context, gate_W, ctx_W, NW):
    T = x_flat.shape[0]
    E = gate_W.shape[0]
    out_shape = [
        jax.ShapeDtypeStruct((T, 1), jnp.int32),
        jax.ShapeDtypeStruct((T, 1), jnp.int32),
        jax.ShapeDtypeStruct((T, 1), jnp.float32),
        jax.ShapeDtypeStruct((T, 1), jnp.float32),
        jax.ShapeDtypeStruct((1, E), jnp.int32),
    ]
    return pl.pallas_call(
        functools.partial(_routing_body, NW=NW), out_shape=out_shape)(
            x_flat, routing_context, gate_W, ctx_W)


def _gmm_body(tile_r, exp_r, lo_r, hi_r, *refs, M, KH):
    x_ref = refs[0]
    b2_ref = refs[1 + 3 * KH]
    ws_ref = refs[2 + 3 * KH]
    o_ref = refs[3 + 3 * KH]
    l = pl.program_id(0)
    x = x_ref[...].astype(jnp.bfloat16)
    y = None
    for i in range(KH):
        w1_ref = refs[1 + 3 * i]
        b1_ref = refs[2 + 3 * i]
        w2_ref = refs[3 + 3 * i]
        h = jnp.dot(x, w1_ref[0].astype(jnp.bfloat16),
                    preferred_element_type=jnp.float32)
        h = h + b1_ref[0, 0][None, :]
        h = 0.5 * h * (1.0 + jax.lax.erf(h * 0.7071067811865476))
        yi = jnp.dot(h.astype(jnp.bfloat16), w2_ref[0].astype(jnp.bfloat16),
                     preferred_element_type=jnp.float32)
        y = yi if y is None else y + yi
    y = y + b2_ref[0, 0][None, :]
    row = tile_r[l] * M + jax.lax.broadcasted_iota(jnp.int32, (M, 1), 0)
    valid = (row >= lo_r[l]) & (row < hi_r[l])
    scale = jnp.where(valid, ws_ref[0, 0][:, None], 0.0)
    contrib = y * scale
    first = jnp.logical_or(l == 0, tile_r[l] != tile_r[jnp.maximum(l - 1, 0)])

    @pl.when(first)
    def _():
        o_ref[...] = contrib

    @pl.when(jnp.logical_not(first))
    def _():
        o_ref[...] += contrib


def _gmm(x_sorted, w_sorted3, W1, b1, W2, b2, tile_of, expert_of, row_lo,
         row_hi, M, KH=4):
    TK, C = x_sorted.shape
    E, _, H = W1.shape
    HC = H // KH
    G = tile_of.shape[0]
    b1r = b1.reshape(E, 1, H)
    b2r = b2.reshape(E, 1, C)
    in_specs = [pl.BlockSpec((M, C), lambda l, t, e, lo, hi: (t[l], 0))]
    operands = [x_sorted]
    for i in range(KH):
        in_specs += [
            pl.BlockSpec((1, C, HC),
                         lambda l, t, e, lo, hi, i=i: (e[l], 0, i)),
            pl.BlockSpec((1, 1, HC),
                         lambda l, t, e, lo, hi, i=i: (e[l], 0, i)),
            pl.BlockSpec((1, HC, C),
                         lambda l, t, e, lo, hi, i=i: (e[l], i, 0)),
        ]
        operands += [W1, b1r, W2]
    in_specs += [
        pl.BlockSpec((1, 1, C), lambda l, t, e, lo, hi: (e[l], 0, 0)),
        pl.BlockSpec((1, 1, M), lambda l, t, e, lo, hi: (t[l], 0, 0)),
    ]
    operands += [b2r, w_sorted3]
    grid_spec = pltpu.PrefetchScalarGridSpec(
        num_scalar_prefetch=4,
        grid=(G,),
        in_specs=in_specs,
        out_specs=pl.BlockSpec((M, C), lambda l, t, e, lo, hi: (t[l], 0)),
    )
    return pl.pallas_call(
        functools.partial(_gmm_body, M=M, KH=KH),
        grid_spec=grid_spec,
        out_shape=jax.ShapeDtypeStruct((TK, C), jnp.float32),
        compiler_params=pltpu.CompilerParams(
            dimension_semantics=("arbitrary",)),
    )(tile_of, expert_of, row_lo, row_hi, *operands)


def _sc_mesh():
    return plsc.VectorSubcoreMesh(core_axis_name="c", subcore_axis_name="s")


def _sc_wid():
    info = plsc.get_sparse_core_info()
    return lax.axis_index("s") * info.num_cores + lax.axis_index("c")


_LANE0 = None


def _lane0():
    return lax.iota(jnp.int32, 16) == 0


def _sc_dispatch(pe, po, wa, wb, x_flat, NW):
    T, C = x_flat.shape
    TK = 2 * T
    TPW = T // NW

    out_type = [
        jax.ShapeDtypeStruct((TK,), jnp.float32),
        jax.ShapeDtypeStruct((TK, C), jnp.float32),
    ]

    @functools.partial(
        pl.kernel,
        out_type=out_type,
        mesh=_sc_mesh(),
        scratch_types=[
            pltpu.VMEM((TPW,), jnp.int32),
            pltpu.VMEM((TPW,), jnp.int32),
            pltpu.VMEM((TPW,), jnp.float32),
            pltpu.VMEM((TPW,), jnp.float32),
            pltpu.VMEM((TPW, C), jnp.float32),
            pltpu.SemaphoreType.DMA,
        ],
    )
    def dispatch_kernel(pe_hbm, po_hbm, wa_hbm, wb_hbm, x_hbm,
                        ws_hbm, xs_hbm,
                        pe_v, po_v, wa_v, wb_v, x_v, sem):
        wid = _sc_wid()
        tbase = wid * TPW
        pltpu.sync_copy(pe_hbm.at[pl.ds(tbase, TPW)], pe_v)
        pltpu.sync_copy(po_hbm.at[pl.ds(tbase, TPW)], po_v)
        pltpu.sync_copy(wa_hbm.at[pl.ds(tbase, TPW)], wa_v)
        pltpu.sync_copy(wb_hbm.at[pl.ds(tbase, TPW)], wb_v)
        pltpu.sync_copy(x_hbm.at[pl.ds(tbase, TPW)], x_v)
        c1 = pltpu.async_copy(wa_v, ws_hbm.at[pe_v], sem)
        c2 = pltpu.async_copy(wb_v, ws_hbm.at[po_v], sem)
        c3 = pltpu.async_copy(x_v, xs_hbm.at[pe_v], sem)
        c4 = pltpu.async_copy(x_v, xs_hbm.at[po_v], sem)
        c1.wait()
        c2.wait()
        c3.wait()
        c4.wait()

    return dispatch_kernel(pe, po, wa, wb, x_flat)


def _sc_combine(y_sorted, pe, po, NW):
    T = pe.shape[0]
    C = y_sorted.shape[1]
    TPW = T // NW

    @functools.partial(
        pl.kernel,
        out_type=jax.ShapeDtypeStruct((T, C), jnp.float32),
        mesh=_sc_mesh(),
        scratch_types=[
            pltpu.VMEM((TPW,), jnp.int32),
            pltpu.VMEM((TPW,), jnp.int32),
            pltpu.VMEM((TPW, C), jnp.float32),
            pltpu.VMEM((TPW, C), jnp.float32),
            pltpu.SemaphoreType.DMA,
        ],
    )
    def combine_kernel(y_hbm, pe_hbm, po_hbm, out_hbm,
                       pe_v, po_v, ya_v, yb_v, sem):
        wid = _sc_wid()
        tbase = wid * TPW
        pltpu.sync_copy(pe_hbm.at[pl.ds(tbase, TPW)], pe_v)
        pltpu.sync_copy(po_hbm.at[pl.ds(tbase, TPW)], po_v)
        ca = pltpu.async_copy(y_hbm.at[pe_v], ya_v, sem)
        cb = pltpu.async_copy(y_hbm.at[po_v], yb_v, sem)
        ca.wait()
        cb.wait()

        def body(r, carry):
            for c in range(C // 16):
                sl = pl.ds(c * 16, 16)
                ya_v[r, sl] = ya_v[r, sl] + yb_v[r, sl]
            return carry

        lax.fori_loop(0, TPW, body, 0)
        pltpu.sync_copy(ya_v, out_hbm.at[pl.ds(tbase, TPW)])

    return combine_kernel(y_sorted, pe, po)


def _block_metadata(offsets, E, NT, M):
    G = NT + E - 1
    counts = offsets[1:] - offsets[:-1]
    t_start = offsets[:-1] // M
    t_end = (offsets[1:] + M - 1) // M
    ntpe = jnp.where(counts > 0, t_end - t_start, 0)
    cum_v = jnp.cumsum(ntpe)
    cum_before = cum_v - ntpe
    l = jnp.arange(G, dtype=jnp.int32)
    eid = jnp.searchsorted(cum_v, l, side="right").astype(jnp.int32)
    eidc = jnp.minimum(eid, E - 1)
    valid = l < cum_v[-1]
    tile_of = jnp.clip(t_start[eidc] + (l - cum_before[eidc]), 0, NT - 1)
    row_lo = jnp.where(valid, offsets[eidc], 0)
    row_hi = jnp.where(valid, offsets[eidc + 1], 0)
    return (tile_of.astype(jnp.int32), eidc.astype(jnp.int32),
            row_lo.astype(jnp.int32), row_hi.astype(jnp.int32))


def kernel(x, routing_context, gate_W, ctx_W, W1, b1, W2, b2):
    B, N, C = x.shape
    E, _, H = W1.shape
    K = 2
    T = B * N
    TK = T * K
    M = 256
    NT = TK // M

    info = plsc.get_sparse_core_info()
    NW = info.num_cores * info.num_subcores

    x_flat = x.reshape(T, C)
    pe, po, wa, wb, counts = _routing(
        x_flat, routing_context, gate_W, ctx_W, NW)
    pe = pe.reshape(T)
    po = po.reshape(T)

    offsets = jnp.concatenate(
        [jnp.zeros((1,), jnp.int32),
         jnp.cumsum(counts[0]).astype(jnp.int32)])
    tile_of, expert_of, row_lo, row_hi = _block_metadata(offsets, E, NT, M)

    ws, x_sorted = _sc_dispatch(
        pe, po, wa.reshape(T), wb.reshape(T), x_flat, NW)

    y_sorted = _gmm(x_sorted, ws.reshape(NT, 1, M), W1, b1, W2, b2,
                    tile_of, expert_of, row_lo, row_hi, M, KH=1)

    out_flat = _sc_combine(y_sorted, pe, po, NW)
    return out_flat.reshape(B, N, C)
